# baseline (speedup 1.0000x reference)
import functools

import jax
import jax.numpy as jnp
from jax import lax
from jax.experimental import pallas as pl
from jax.experimental.pallas import tpu as pltpu
from jax.experimental.pallas import tpu_sc as plsc

_NC = 2
_NS = 16
_NW = _NC * _NS

_D = 64
_NR = 32
_SCHUNK = 512
_WAVE = 16


def _sc_gather(table, idx):
    b3 = idx.shape[0]
    b_per_w = b3 // _NW
    n_stages = b_per_w // _SCHUNK
    assert b_per_w % _SCHUNK == 0 and _SCHUNK % _WAVE == 0

    mesh = plsc.VectorSubcoreMesh(core_axis_name="c", subcore_axis_name="s")

    @functools.partial(
        pl.kernel,
        out_type=jax.ShapeDtypeStruct((b3, _D), jnp.float32),
        mesh=mesh,
        compiler_params=pltpu.CompilerParams(use_tc_tiling_on_sc=True),
        scratch_types=[
            pltpu.VMEM((_SCHUNK,), jnp.int32),
            pltpu.SemaphoreType.DMA,
        ],
    )
    def gather_kernel(table_hbm, idx_hbm, out_hbm, idx_v, sem):
        wid = lax.axis_index("s") * _NC + lax.axis_index("c")
        base = wid * b_per_w

        for stage in range(n_stages):
            sbase = base + stage * _SCHUNK
            pltpu.sync_copy(idx_hbm.at[pl.ds(sbase, _SCHUNK)], idx_v)

            def wave_body(i, sbase=sbase):
                v = idx_v[pl.ds(i * _WAVE, _WAVE)]
                for j in range(_WAVE):
                    e = v[j]
                    pltpu.make_async_copy(
                        table_hbm.at[pl.ds(e, 1)],
                        out_hbm.at[pl.ds(sbase + i * _WAVE + j, 1)],
                        sem,
                    ).start()

            lax.fori_loop(0, _SCHUNK // _WAVE,
                          lambda i, c: (wave_body(i), c)[1], 0)
            pltpu.make_async_copy(
                table_hbm.at[pl.ds(0, _SCHUNK)],
                out_hbm.at[pl.ds(sbase, _SCHUNK)],
                sem,
            ).wait()

    return gather_kernel(table, idx)


def _tc_transform(r2d, rows3, w_flat, rel_embed, batch, blk):
    n_blocks = batch // blk
    kdim = _NR * _D

    def body(r_ref, xh_ref, xp_ref, xn_ref, wf_ref, rel_ref,
             oh_ref, op_ref, on_ref, or_ref):
        rcol = r_ref[...]
        lane_rel = lax.broadcasted_iota(jnp.int32, (blk, kdim), 1) >> 6
        mask = lane_rel == rcol
        wf = wf_ref[...]

        def trans(x_ref, o_ref):
            x = x_ref[...]
            xt = jnp.concatenate([x] * _NR, axis=1)
            z = jnp.where(mask, xt, 0.0)
            o_ref[...] = jnp.dot(z, wf, preferred_element_type=jnp.float32)

        trans(xh_ref, oh_ref)
        trans(xp_ref, op_ref)
        trans(xn_ref, on_ref)

        onehot = (lax.broadcasted_iota(jnp.int32, (blk, _NR), 1)
                  == rcol).astype(jnp.float32)
        or_ref[...] = jnp.dot(onehot, rel_ref[...],
                              preferred_element_type=jnp.float32)

    out_block = pl.BlockSpec((blk, _D), lambda i: (i, 0))
    return pl.pallas_call(
        body,
        grid=(n_blocks,),
        in_specs=[
            pl.BlockSpec((blk, 1), lambda i: (i, 0)),
            pl.BlockSpec((blk, _D), lambda i: (i, 0)),
            pl.BlockSpec((blk, _D), lambda i: (i + n_blocks, 0)),
            pl.BlockSpec((blk, _D), lambda i: (i + 2 * n_blocks, 0)),
            pl.BlockSpec((kdim, _D), lambda i: (0, 0)),
            pl.BlockSpec((_NR, _D), lambda i: (0, 0)),
        ],
        out_specs=[out_block, out_block, out_block, out_block],
        out_shape=[jax.ShapeDtypeStruct((batch, _D), jnp.float32)] * 4,
    )(r2d, rows3, rows3, rows3, w_flat, rel_embed)


def kernel(h, r, pos_t, neg_t, entity_embed, relation_embed, W_r):
    batch = h.shape[0]
    idx_all = jnp.concatenate([h, pos_t, neg_t]).astype(jnp.int32)
    rows3 = _sc_gather(entity_embed, idx_all)
    w_flat = W_r.reshape(_NR * _D, _D)
    r2d = r.astype(jnp.int32)[:, None]
    h_e, pos_t_e, neg_t_e, r_embed = _tc_transform(
        r2d, rows3, w_flat, relation_embed, batch, blk=512)
    return (h_e, pos_t_e, neg_t_e, r_embed)


# speedup vs baseline: 1.8160x; 1.8160x over previous
import functools

import jax
import jax.numpy as jnp
from jax import lax
from jax.experimental import pallas as pl
from jax.experimental.pallas import tpu as pltpu
from jax.experimental.pallas import tpu_sc as plsc

_NC = 2
_NS = 16
_NW = _NC * _NS

_D = 64
_NR = 32
_CHUNK = 128


def _sc_gather128(table128, idx):
    b3 = idx.shape[0]
    b_per_w = b3 // _NW
    n_q = 4
    q_rows = b_per_w // n_q
    n_chunks = q_rows // _CHUNK
    assert q_rows % _CHUNK == 0

    mesh = plsc.VectorSubcoreMesh(core_axis_name="c", subcore_axis_name="s")

    @functools.partial(
        pl.kernel,
        out_type=jax.ShapeDtypeStruct((b3, 128), jnp.float32),
        mesh=mesh,
        compiler_params=pltpu.CompilerParams(use_tc_tiling_on_sc=True),
        scratch_types=[
            pltpu.VMEM((b_per_w,), jnp.int32),
            pltpu.VMEM((q_rows, 128), jnp.float32),
            pltpu.VMEM((q_rows, 128), jnp.float32),
            pltpu.SemaphoreType.DMA,
            pltpu.SemaphoreType.DMA,
        ],
    )
    def gather_kernel(table_hbm, idx_hbm, out_hbm, idx_v, rows_a, rows_b,
                      sem_a, sem_b):
        wid = lax.axis_index("s") * _NC + lax.axis_index("c")
        base = wid * b_per_w
        pltpu.sync_copy(idx_hbm.at[pl.ds(base, b_per_w)], idx_v)
        bufs = ((rows_a, sem_a), (rows_b, sem_b))

        def chunk_copies(qq):
            rows_v, sem = bufs[qq % 2]
            return [
                pltpu.make_async_copy(
                    table_hbm.at[idx_v.at[pl.ds(qq * q_rows + j * _CHUNK,
                                                _CHUNK)]],
                    rows_v.at[pl.ds(j * _CHUNK, _CHUNK)],
                    sem,
                )
                for j in range(n_chunks)
            ]

        for c in chunk_copies(0):
            c.start()
        for qq in range(1, n_q + 1):
            if qq <= n_q - 1:
                for c in chunk_copies(qq):
                    c.start()
            prev = qq - 1
            rows_v, _ = bufs[prev % 2]
            for c in chunk_copies(prev):
                c.wait()
            pltpu.sync_copy(rows_v,
                            out_hbm.at[pl.ds(base + prev * q_rows, q_rows)])

    return gather_kernel(table128, idx)


def _tc_transform(r2d, rows3, w_flat, rel_embed, batch, blk):
    n_blocks = batch // blk
    kdim = _NR * _D

    def body(r_ref, xh_ref, xp_ref, xn_ref, wf_ref, rel_ref,
             oh_ref, op_ref, on_ref, or_ref):
        rcol = r_ref[...]
        lane_rel = lax.broadcasted_iota(jnp.int32, (blk, kdim), 1) >> 6
        mask = lane_rel == rcol
        wf = wf_ref[...]

        def trans(x_ref, o_ref):
            x = x_ref[...][:, :_D]
            xt = jnp.concatenate([x] * _NR, axis=1)
            z = jnp.where(mask, xt, 0.0)
            o_ref[...] = jnp.dot(z, wf, preferred_element_type=jnp.float32)

        trans(xh_ref, oh_ref)
        trans(xp_ref, op_ref)
        trans(xn_ref, on_ref)

        onehot = (lax.broadcasted_iota(jnp.int32, (blk, _NR), 1)
                  == rcol).astype(jnp.float32)
        or_ref[...] = jnp.dot(onehot, rel_ref[...],
                              preferred_element_type=jnp.float32)

    out_block = pl.BlockSpec((blk, _D), lambda i: (i, 0))
    return pl.pallas_call(
        body,
        grid=(n_blocks,),
        in_specs=[
            pl.BlockSpec((blk, 1), lambda i: (i, 0)),
            pl.BlockSpec((blk, 128), lambda i: (i, 0)),
            pl.BlockSpec((blk, 128), lambda i: (i + n_blocks, 0)),
            pl.BlockSpec((blk, 128), lambda i: (i + 2 * n_blocks, 0)),
            pl.BlockSpec((kdim, _D), lambda i: (0, 0)),
            pl.BlockSpec((_NR, _D), lambda i: (0, 0)),
        ],
        out_specs=[out_block, out_block, out_block, out_block],
        out_shape=[jax.ShapeDtypeStruct((batch, _D), jnp.float32)] * 4,
    )(r2d, rows3, rows3, rows3, w_flat, rel_embed)


def kernel(h, r, pos_t, neg_t, entity_embed, relation_embed, W_r):
    batch = h.shape[0]
    table128 = jnp.pad(entity_embed, ((0, 0), (0, 128 - _D)))
    idx_all = jnp.concatenate([h, pos_t, neg_t]).astype(jnp.int32)
    rows3 = _sc_gather128(table128, idx_all)
    w_flat = W_r.reshape(_NR * _D, _D)
    r2d = r.astype(jnp.int32)[:, None]
    h_e, pos_t_e, neg_t_e, r_embed = _tc_transform(
        r2d, rows3, w_flat, relation_embed, batch, blk=512)
    return (h_e, pos_t_e, neg_t_e, r_embed)
